# baseline (speedup 1.0000x reference)
import functools

import jax
import jax.numpy as jnp
from jax import lax
from jax.experimental import pallas as pl
from jax.experimental.pallas import tpu as pltpu
from jax.experimental.pallas import tpu_sc as plsc

D_MODEL = 128
B_PAD = 136
ROWS_PER_TILE = 8
NUM_ACTIVE = B_PAD // ROWS_PER_TILE
NUM_TABLES = 6
LANES = 16


def _sc_body(xt_hbm, t0, t1, t2, t3, t4, t5, out_hbm, idx_v, gath_v, acc_v, sem):
    cid = lax.axis_index("c")
    sid = lax.axis_index("s")
    wid = sid * 2 + cid

    @pl.when(wid < NUM_ACTIVE)
    def _():
        base = wid * ROWS_PER_TILE
        for t in range(NUM_TABLES):
            pltpu.sync_copy(
                xt_hbm.at[pl.ds(t * B_PAD + base, ROWS_PER_TILE)], idx_v.at[t]
            )
        tables = (t0, t1, t2, t3, t4, t5)
        copies = []
        for t in range(NUM_TABLES):
            copies.append(
                pltpu.async_copy(tables[t].at[idx_v.at[t]], gath_v.at[t], sem)
            )
        for cp in copies:
            cp.wait()
        for i in range(ROWS_PER_TILE):
            for c in range(D_MODEL // LANES):
                sl = pl.ds(c * LANES, LANES)
                acc_v[i, sl] = (
                    gath_v[0, i, sl]
                    + gath_v[1, i, sl]
                    + gath_v[2, i, sl]
                    + gath_v[3, i, sl]
                    + gath_v[4, i, sl]
                    + gath_v[5, i, sl]
                )
        pltpu.sync_copy(acc_v, out_hbm.at[pl.ds(base, ROWS_PER_TILE)])


@jax.jit
def _sc_embed(xt, turn_table, card_table, action_table, pos_table, civ_table,
              face_table):
    mesh = plsc.VectorSubcoreMesh(core_axis_name="c", subcore_axis_name="s")
    return pl.kernel(
        _sc_body,
        out_type=jax.ShapeDtypeStruct((B_PAD, D_MODEL), jnp.float32),
        mesh=mesh,
        scratch_types=[
            pltpu.VMEM((NUM_TABLES, ROWS_PER_TILE), jnp.int32),
            pltpu.VMEM((NUM_TABLES, ROWS_PER_TILE, D_MODEL), jnp.float32),
            pltpu.VMEM((ROWS_PER_TILE, D_MODEL), jnp.float32),
            pltpu.SemaphoreType.DMA,
        ],
    )(xt, turn_table, card_table, action_table, pos_table, civ_table,
      face_table)


def kernel(x, turn_table, card_table, action_table, pos_table, civ_table,
           face_table):
    b = x.shape[0]
    xt = jnp.transpose(x.astype(jnp.int32))
    xt = jnp.pad(xt, ((0, 0), (0, B_PAD - b)))
    xt = jnp.reshape(xt, (NUM_TABLES * B_PAD,))
    out = _sc_embed(xt, turn_table, card_table, action_table, pos_table,
                    civ_table, face_table)
    return out[:b]
